# Initial kernel scaffold; baseline (speedup 1.0000x reference)
import jax
import jax.numpy as jnp
from jax.experimental import pallas as pl


def kernel(x, adj, g, dropout, W_tag0, b_tag0, W_tag1, b_tag1, W_gin, b_gin, eps_gin, W_gcn, b_gcn):
    raise NotImplementedError("write your pallas kernel here")


# speedup vs baseline: 1.0772x; 1.0772x over previous
import functools

import jax
import jax.numpy as jnp
from jax.experimental import pallas as pl
from jax.experimental.pallas import tpu as pltpu

N, D = 4096, 128
BM = 256


def _fused_kernel(eps_ref, adj_ref, x_ref, g_ref, w0_ref, w1x_ref, w1a_ref,
                  wgin_ref, wgcn_ref, bmat_ref, out_ref):
    i = pl.program_id(0)
    xt = x_ref[pl.ds(i * BM, BM), :]
    ax = jnp.dot(adj_ref[...], x_ref[...], preferred_element_type=jnp.float32)
    gv = g_ref[...]
    eps = eps_ref[0]
    u = (1.0 + eps) * xt + ax
    out = (gv[:, 0:1] * jnp.dot(xt, w0_ref[...], preferred_element_type=jnp.float32)
           + gv[:, 1:2] * (jnp.dot(xt, w1x_ref[...], preferred_element_type=jnp.float32)
                           + jnp.dot(ax, w1a_ref[...], preferred_element_type=jnp.float32))
           + gv[:, 2:3] * jnp.dot(u, wgin_ref[...], preferred_element_type=jnp.float32)
           + gv[:, 3:4] * jnp.dot(ax, wgcn_ref[...], preferred_element_type=jnp.float32)
           + jnp.dot(gv, bmat_ref[...], preferred_element_type=jnp.float32))
    out_ref[...] = out


@functools.partial(jax.jit, static_argnames=("interpret",))
def _run(x, adj, g, eps_gin, W_tag0, W_tag1, W_gin, W_gcn, bmat,
         interpret=False):
    eps = jnp.asarray(eps_gin, jnp.float32).reshape((1,))
    W1x = W_tag1[:D, :]
    W1a = W_tag1[D:, :]
    full = lambda shape: pl.BlockSpec(shape, lambda i: (0, 0))
    return pl.pallas_call(
        _fused_kernel,
        grid=(N // BM,),
        in_specs=[
            pl.BlockSpec(memory_space=pltpu.SMEM),
            pl.BlockSpec((BM, N), lambda i: (i, 0)),
            full((N, D)),
            pl.BlockSpec((BM, 4), lambda i: (i, 0)),
            full((D, D)), full((D, D)), full((D, D)),
            full((D, D)), full((D, D)),
            full((4, D)),
        ],
        out_specs=pl.BlockSpec((BM, D), lambda i: (i, 0)),
        out_shape=jax.ShapeDtypeStruct((N, D), jnp.float32),
        interpret=interpret,
    )(eps, adj, x, g, W_tag0, W1x, W1a, W_gin, W_gcn, bmat)


def kernel(x, adj, g, dropout, W_tag0, b_tag0, W_tag1, b_tag1, W_gin, b_gin,
           eps_gin, W_gcn, b_gcn):
    bmat = jnp.stack([b_tag0, b_tag1, b_gin, b_gcn], axis=0)
    return _run(x, adj, g, eps_gin, W_tag0, W_tag1, W_gin, W_gcn, bmat)


# speedup vs baseline: 1.1167x; 1.0367x over previous
import functools

import jax
import jax.numpy as jnp
from jax.experimental import pallas as pl
from jax.experimental.pallas import tpu as pltpu

N, D = 4096, 128
BM = 256


def _fused_kernel(eps_ref, adj_ref, x_ref, g_ref, w0_ref, w1x_ref, w1a_ref,
                  wgin_ref, wgcn_ref, bmat_ref, out_ref):
    i = pl.program_id(0)
    xt = x_ref[pl.ds(i * BM, BM), :]
    ax = jnp.dot(adj_ref[...].astype(jnp.bfloat16),
                 x_ref[...].astype(jnp.bfloat16),
                 preferred_element_type=jnp.float32)
    gv = g_ref[...]
    eps = eps_ref[0]
    u = (1.0 + eps) * xt + ax
    out = (gv[:, 0:1] * jnp.dot(xt, w0_ref[...], preferred_element_type=jnp.float32)
           + gv[:, 1:2] * (jnp.dot(xt, w1x_ref[...], preferred_element_type=jnp.float32)
                           + jnp.dot(ax, w1a_ref[...], preferred_element_type=jnp.float32))
           + gv[:, 2:3] * jnp.dot(u, wgin_ref[...], preferred_element_type=jnp.float32)
           + gv[:, 3:4] * jnp.dot(ax, wgcn_ref[...], preferred_element_type=jnp.float32)
           + jnp.dot(gv, bmat_ref[...], preferred_element_type=jnp.float32))
    out_ref[...] = out


@functools.partial(jax.jit, static_argnames=("interpret",))
def _run(x, adj, g, eps_gin, W_tag0, W_tag1, W_gin, W_gcn, bmat,
         interpret=False):
    eps = jnp.asarray(eps_gin, jnp.float32).reshape((1,))
    W1x = W_tag1[:D, :]
    W1a = W_tag1[D:, :]
    full = lambda shape: pl.BlockSpec(shape, lambda i: (0, 0))
    return pl.pallas_call(
        _fused_kernel,
        grid=(N // BM,),
        in_specs=[
            pl.BlockSpec(memory_space=pltpu.SMEM),
            pl.BlockSpec((BM, N), lambda i: (i, 0)),
            full((N, D)),
            pl.BlockSpec((BM, 4), lambda i: (i, 0)),
            full((D, D)), full((D, D)), full((D, D)),
            full((D, D)), full((D, D)),
            full((4, D)),
        ],
        out_specs=pl.BlockSpec((BM, D), lambda i: (i, 0)),
        out_shape=jax.ShapeDtypeStruct((N, D), jnp.float32),
        interpret=interpret,
    )(eps, adj, x, g, W_tag0, W1x, W1a, W_gin, W_gcn, bmat)


def kernel(x, adj, g, dropout, W_tag0, b_tag0, W_tag1, b_tag1, W_gin, b_gin,
           eps_gin, W_gcn, b_gcn):
    bmat = jnp.stack([b_tag0, b_tag1, b_gin, b_gcn], axis=0)
    return _run(x, adj, g, eps_gin, W_tag0, W_tag1, W_gin, W_gcn, bmat)


# speedup vs baseline: 1.1802x; 1.0568x over previous
import functools

import jax
import jax.numpy as jnp
from jax.experimental import pallas as pl
from jax.experimental.pallas import tpu as pltpu

N, D = 4096, 128
BM = 512


def _fused_kernel(eps_ref, adj_ref, x_ref, g_ref, w0_ref, w1x_ref, w1a_ref,
                  wgin_ref, wgcn_ref, bmat_ref, out_ref):
    i = pl.program_id(0)
    xt = x_ref[pl.ds(i * BM, BM), :]
    ax = jnp.dot(adj_ref[...].astype(jnp.bfloat16),
                 x_ref[...].astype(jnp.bfloat16),
                 preferred_element_type=jnp.float32)
    gv = g_ref[...]
    eps = eps_ref[0]
    u = (1.0 + eps) * xt + ax
    out = (gv[:, 0:1] * jnp.dot(xt, w0_ref[...], preferred_element_type=jnp.float32)
           + gv[:, 1:2] * (jnp.dot(xt, w1x_ref[...], preferred_element_type=jnp.float32)
                           + jnp.dot(ax, w1a_ref[...], preferred_element_type=jnp.float32))
           + gv[:, 2:3] * jnp.dot(u, wgin_ref[...], preferred_element_type=jnp.float32)
           + gv[:, 3:4] * jnp.dot(ax, wgcn_ref[...], preferred_element_type=jnp.float32)
           + jnp.dot(gv, bmat_ref[...], preferred_element_type=jnp.float32))
    out_ref[...] = out


@functools.partial(jax.jit, static_argnames=("interpret",))
def _run(x, adj, g, eps_gin, W_tag0, W_tag1, W_gin, W_gcn, bmat,
         interpret=False):
    eps = jnp.asarray(eps_gin, jnp.float32).reshape((1,))
    W1x = W_tag1[:D, :]
    W1a = W_tag1[D:, :]
    full = lambda shape: pl.BlockSpec(shape, lambda i: (0, 0))
    return pl.pallas_call(
        _fused_kernel,
        grid=(N // BM,),
        in_specs=[
            pl.BlockSpec(memory_space=pltpu.SMEM),
            pl.BlockSpec((BM, N), lambda i: (i, 0)),
            full((N, D)),
            pl.BlockSpec((BM, 4), lambda i: (i, 0)),
            full((D, D)), full((D, D)), full((D, D)),
            full((D, D)), full((D, D)),
            full((4, D)),
        ],
        out_specs=pl.BlockSpec((BM, D), lambda i: (i, 0)),
        out_shape=jax.ShapeDtypeStruct((N, D), jnp.float32),
        interpret=interpret,
    )(eps, adj, x, g, W_tag0, W1x, W1a, W_gin, W_gcn, bmat)


def kernel(x, adj, g, dropout, W_tag0, b_tag0, W_tag1, b_tag1, W_gin, b_gin,
           eps_gin, W_gcn, b_gcn):
    bmat = jnp.stack([b_tag0, b_tag1, b_gin, b_gcn], axis=0)
    return _run(x, adj, g, eps_gin, W_tag0, W_tag1, W_gin, W_gcn, bmat)
